# baseline (device time: 168527 ns/iter reference)
import jax
import jax.numpy as jnp
from jax import lax
from jax.experimental import pallas as pl
from jax.experimental.pallas import tpu as pltpu

N_Z = 4
P = 8
NP = (N_Z - 1) * P
NSLOT = 3
N_FSITES = 24


def kernel(x):
    m_per, n = x.shape
    Q = m_per // 4
    Rq = Q // P

    def body(x_ref, out_ref, comm_ref, stage_ref, *sems):
        (s_zr, r_zl, s_zl, r_zr,
         s_xd0, r_xd0, s_xd1, r_xd1,
         s_yd0, r_yd0, s_yd1, r_yd1,
         s_yr0, r_yr0, s_yr1, r_yr1,
         s_xr0, r_xr0, s_xr1, r_xr1, dsem) = sems
        s_xd = (s_xd0, s_xd1); r_xd = (r_xd0, r_xd1)
        s_yd = (s_yd0, s_yd1); r_yd = (r_yd0, r_yd1)
        s_yr = (s_yr0, s_yr1); r_yr = (r_yr0, r_yr1)
        s_xr = (s_xr0, s_xr1); r_xr = (r_xr0, r_xr1)

        my_x = lax.axis_index("x")
        my_y = lax.axis_index("y")
        my_z = lax.axis_index("z")
        yi = my_y % 2
        mate_y = my_y + 1 - 2 * yi
        k = 2 * my_x + yi
        kx = 2 * (1 - my_x) + yi
        ky = 2 * my_x + (1 - yi)
        kd = 3 - k

        dev_self = (my_x, my_y, my_z)
        dev_xp = (1 - my_x, my_y, my_z)
        dev_ym = (my_x, mate_y, my_z)

        def prow(c, qk, i):
            return pl.ds(c * m_per + qk * Q + (i % P) * Rq, Rq)

        def qrow(c, qk):
            return pl.ds(c * m_per + qk * Q, Q)

        def c_of(d, i):
            return (my_z - 1 - i // P) if d == 0 else (my_z + 1 + i // P)

        def R_of(d, i):
            if d == 0:
                return my_z >= 1 + i // P
            return my_z + 1 + i // P <= N_Z - 1

        def S_R(i):
            return (my_z <= N_Z - 2) & (my_z >= i // P)

        def S_L(i):
            return (my_z >= 1) & (my_z + i // P <= N_Z - 1)

        fsites = []

        def _fsite_cq(kind, d, j):
            qk = {"k": k, "kx": kx, "ky": ky, "kd": kd}[kind]
            return c_of(d, j), qk

        def fdma(idx):
            kind, d, j = fsites[idx]
            c, qk = _fsite_cq(kind, d, j)
            return pltpu.make_async_copy(
                stage_ref.at[idx % NSLOT],
                out_ref.at[qrow(c, qk), :],
                dsem.at[idx],
            )

        def hbm_store(kind, d, j):
            idx = len(fsites)
            if idx >= NSLOT:
                pk, pd, pj = fsites[idx - NSLOT]

                @pl.when(R_of(pd, pj))
                def _(pidx=idx - NSLOT):
                    fdma(pidx).wait()

            fsites.append((kind, d, j))

            @pl.when(R_of(d, j))
            def _(idx=idx, kind=kind, d=d, j=j):
                c, qk = _fsite_cq(kind, d, j)
                stage_ref[idx % NSLOT, :, :] = comm_ref[
                    qrow(c, qk), :].astype(jnp.float32)
                fdma(idx).start()

        def mk(rowslice, ssem, rsem, i, dev):
            return pltpu.make_async_remote_copy(
                src_ref=comm_ref.at[rowslice, :],
                dst_ref=comm_ref.at[rowslice, :],
                send_sem=ssem.at[i],
                recv_sem=rsem.at[i],
                device_id=dev,
                device_id_type=pl.DeviceIdType.MESH,
            )

        def send_zr(i):
            return mk(prow(my_z - i // P, k, i), s_zr, r_zl, i,
                      (my_x, my_y, my_z + 1))

        def send_zl(i):
            return mk(prow(my_z + i // P, k, i), s_zl, r_zr, i,
                      (my_x, my_y, my_z - 1))

        def recv_z(d, i):
            ssem, rsem = (s_zr, r_zl) if d == 0 else (s_zl, r_zr)
            return mk(prow(c_of(d, i), k, i), ssem, rsem, i, dev_self)

        def xd_send(d, i):
            return mk(prow(c_of(d, i), k, i), s_xd[d], r_xd[d], i, dev_xp)

        def xd_recv(d, i):
            return mk(prow(c_of(d, i), kx, i), s_xd[d], r_xd[d], i, dev_self)

        def yd_send(d, i):
            return mk(prow(c_of(d, i), k, i), s_yd[d], r_yd[d], i, dev_ym)

        def yd_recv(d, i):
            return mk(prow(c_of(d, i), ky, i), s_yd[d], r_yd[d], i, dev_self)

        def yrel_send(d, i):
            return mk(prow(c_of(d, i), kx, i), s_yr[d], r_yr[d], i, dev_ym)

        def yrel_recv(d, i):
            return mk(prow(c_of(d, i), kd, i), s_yr[d], r_yr[d], i, dev_self)

        def xrel_send(d, i):
            return mk(prow(c_of(d, i), ky, i), s_xr[d], r_xr[d], i, dev_xp)

        def xrel_recv(d, i):
            return mk(prow(c_of(d, i), kd, i), s_xr[d], r_xr[d], i, dev_self)

        barrier = pltpu.get_barrier_semaphore()

        @pl.when(my_z >= 1)
        def _():
            pl.semaphore_signal(
                barrier, inc=1, device_id=(my_x, my_y, my_z - 1),
                device_id_type=pl.DeviceIdType.MESH)

        @pl.when(my_z <= N_Z - 2)
        def _():
            pl.semaphore_signal(
                barrier, inc=1, device_id=(my_x, my_y, my_z + 1),
                device_id_type=pl.DeviceIdType.MESH)

        pl.semaphore_signal(barrier, inc=1, device_id=dev_xp,
                            device_id_type=pl.DeviceIdType.MESH)
        pl.semaphore_signal(barrier, inc=1, device_id=dev_ym,
                            device_id_type=pl.DeviceIdType.MESH)
        n_nbrs = 2 + (my_z >= 1).astype(jnp.int32) + (
            my_z <= N_Z - 2).astype(jnp.int32)
        pl.semaphore_wait(barrier, n_nbrs)

        own_dma = pltpu.make_async_copy(
            x_ref, out_ref.at[pl.ds(my_z * m_per, m_per), :],
            dsem.at[N_FSITES])
        own_dma.start()

        for i in range(P):
            comm_ref[prow(my_z, k, i), :] = x_ref[
                pl.ds(k * Q + (i % P) * Rq, Rq), :].astype(comm_ref.dtype)

            @pl.when(S_R(i))
            def _(i=i):
                send_zr(i).start()

            @pl.when(S_L(i))
            def _(i=i):
                send_zl(i).start()

        for i in range(P, NP):
            j = i - P

            @pl.when(R_of(0, j))
            def _(j=j):
                recv_z(0, j).wait_recv()
                xd_send(0, j).start()
                yd_send(0, j).start()

            @pl.when(S_R(i))
            def _(i=i):
                send_zr(i).start()

            @pl.when(R_of(1, j))
            def _(j=j):
                recv_z(1, j).wait_recv()
                xd_send(1, j).start()
                yd_send(1, j).start()

            @pl.when(S_L(i))
            def _(i=i):
                send_zl(i).start()

            if j % P == P - 1:
                hbm_store("k", 0, j)
                hbm_store("k", 1, j)

            j2 = i - 2 * P
            if j2 >= 0:
                for d in (0, 1):
                    @pl.when(R_of(d, j2))
                    def _(d=d, j2=j2):
                        xd_recv(d, j2).wait_recv()
                        if j2 % 2 == 0:
                            yrel_send(d, j2).start()

                    if j2 % P == P - 1:
                        hbm_store("kx", d, j2)

                    @pl.when(R_of(d, j2))
                    def _(d=d, j2=j2):
                        yd_recv(d, j2).wait_recv()
                        if j2 % 2 == 1:
                            xrel_send(d, j2).start()

                    if j2 % P == P - 1:
                        hbm_store("ky", d, j2)

        for j in range(NP - P, NP):
            for d in (0, 1):
                @pl.when(R_of(d, j))
                def _(d=d, j=j):
                    recv_z(d, j).wait_recv()
                    xd_send(d, j).start()
                    yd_send(d, j).start()

            if j % P == P - 1:
                hbm_store("k", 0, j)
                hbm_store("k", 1, j)

        for j2 in range(max(NP - 2 * P, 0), NP):
            for d in (0, 1):
                @pl.when(R_of(d, j2))
                def _(d=d, j2=j2):
                    xd_recv(d, j2).wait_recv()
                    if j2 % 2 == 0:
                        yrel_send(d, j2).start()

                if j2 % P == P - 1:
                    hbm_store("kx", d, j2)

                @pl.when(R_of(d, j2))
                def _(d=d, j2=j2):
                    yd_recv(d, j2).wait_recv()
                    if j2 % 2 == 1:
                        xrel_send(d, j2).start()

                if j2 % P == P - 1:
                    hbm_store("ky", d, j2)

        for i in range(NP):
            for d in (0, 1):
                if i % 2 == 0:
                    @pl.when(R_of(d, i))
                    def _(d=d, i=i):
                        yrel_recv(d, i).wait_recv()
                else:
                    @pl.when(R_of(d, i))
                    def _(d=d, i=i):
                        xrel_recv(d, i).wait_recv()

                    if i % P == P - 1:
                        hbm_store("kd", d, i)

        assert len(fsites) == N_FSITES, len(fsites)
        for idx in range(N_FSITES - NSLOT, N_FSITES):
            kind, d, j = fsites[idx]

            @pl.when(R_of(d, j))
            def _(idx=idx):
                fdma(idx).wait()

        own_dma.wait()

        for i in range(NP):
            @pl.when(S_R(i))
            def _(i=i):
                send_zr(i).wait_send()

            @pl.when(S_L(i))
            def _(i=i):
                send_zl(i).wait_send()

            for d in (0, 1):
                @pl.when(R_of(d, i))
                def _(d=d, i=i):
                    xd_send(d, i).wait_send()
                    yd_send(d, i).wait_send()
                    if i % 2 == 0:
                        yrel_send(d, i).wait_send()
                    else:
                        xrel_send(d, i).wait_send()

    return pl.pallas_call(
        body,
        out_shape=jax.ShapeDtypeStruct((N_Z * m_per, n), jnp.float32),
        in_specs=[pl.BlockSpec(memory_space=pltpu.VMEM)],
        out_specs=pl.BlockSpec(memory_space=pl.MemorySpace.ANY),
        scratch_shapes=(
            [pltpu.VMEM((N_Z * m_per, n), jnp.bfloat16),
             pltpu.VMEM((NSLOT, m_per // 4, n), jnp.float32)]
            + [pltpu.SemaphoreType.DMA((NP,)) for _ in range(20)]
            + [pltpu.SemaphoreType.DMA((N_FSITES + 1,))]
        ),
        compiler_params=pltpu.CompilerParams(
            collective_id=0, vmem_limit_bytes=100 * 1024 * 1024),
    )(x)


# device time: 156047 ns/iter; 1.0800x vs baseline; 1.0800x over previous
import jax
import jax.numpy as jnp
from jax import lax
from jax.experimental import pallas as pl
from jax.experimental.pallas import tpu as pltpu

N_Z = 4
P = 8
NP = (N_Z - 1) * P
P2 = 3
NP2 = (N_Z - 1) * P2
B_SET = (3, 4)
C_SET = (5, 6, 7)
N_SITES = 28


def kernel(x):
    m_per, n = x.shape
    Q = m_per // 4
    Rq = Q // P

    def body(x_ref, out_ref, comm_ref, *sems):
        (s_zr, r_zl, s_zl, r_zr,
         s_z2r, r_z2l, s_z2l, r_z2r,
         s_xd0, r_xd0, s_xd1, r_xd1,
         s_yd0, r_yd0, s_yd1, r_yd1,
         s_yr0, r_yr0, s_yr1, r_yr1,
         s_xr0, r_xr0, s_xr1, r_xr1, dsem) = sems
        s_xd = (s_xd0, s_xd1); r_xd = (r_xd0, r_xd1)
        s_yd = (s_yd0, s_yd1); r_yd = (r_yd0, r_yd1)
        s_yr = (s_yr0, s_yr1); r_yr = (r_yr0, r_yr1)
        s_xr = (s_xr0, s_xr1); r_xr = (r_xr0, r_xr1)

        my_x = lax.axis_index("x")
        my_y = lax.axis_index("y")
        my_z = lax.axis_index("z")
        yi = my_y % 2
        mate_y = my_y + 1 - 2 * yi
        k = 2 * my_x + yi
        kx = 2 * (1 - my_x) + yi
        ky = 2 * my_x + (1 - yi)
        kd = 3 - k

        dev_self = (my_x, my_y, my_z)
        dev_xp = (1 - my_x, my_y, my_z)
        dev_ym = (my_x, mate_y, my_z)

        def prow(c, qk, p):
            return pl.ds(c * m_per + qk * Q + (p % P) * Rq, Rq)

        def qrow(c, qk):
            return pl.ds(c * m_per + qk * Q, Q)

        def c_of(d, i):
            return (my_z - 1 - i // P) if d == 0 else (my_z + 1 + i // P)

        def R_of(d, i):
            if d == 0:
                return my_z >= 1 + i // P
            return my_z + 1 + i // P <= N_Z - 1

        def S_R(i):
            return (my_z <= N_Z - 2) & (my_z >= i // P)

        def S_L(i):
            return (my_z >= 1) & (my_z + i // P <= N_Z - 1)

        def c2_of(d, i2):
            return (my_z - 1 - i2 // P2) if d == 0 else (my_z + 1 + i2 // P2)

        def R2_of(d, i2):
            if d == 0:
                return my_z >= 1 + i2 // P2
            return my_z + 1 + i2 // P2 <= N_Z - 1

        def S2_R(i2):
            return (my_z <= N_Z - 2) & (my_z >= i2 // P2)

        def S2_L(i2):
            return (my_z >= 1) & (my_z + i2 // P2 <= N_Z - 1)

        sites = []

        def _site_cq(kind, a, j):
            if kind == "own":
                return my_z, (k + a) % 4
            qk = {"k": k, "kx": kx, "ky": ky, "kd": kd}[kind]
            return c_of(a, j), qk

        def hbm_copy(idx, kind, a, j):
            c, qk = _site_cq(kind, a, j)
            return pltpu.make_async_copy(
                comm_ref.at[qrow(c, qk), :],
                out_ref.at[qrow(c, qk), :],
                dsem.at[idx],
            )

        def hbm_store(kind, a, j=0):
            hbm_copy(len(sites), kind, a, j).start()
            sites.append((kind, a, j))

        def mk(rowslice, ssem, rsem, i, dev):
            return pltpu.make_async_remote_copy(
                src_ref=comm_ref.at[rowslice, :],
                dst_ref=comm_ref.at[rowslice, :],
                send_sem=ssem.at[i],
                recv_sem=rsem.at[i],
                device_id=dev,
                device_id_type=pl.DeviceIdType.MESH,
            )

        def send_zr(i):
            return mk(prow(my_z - i // P, k, i), s_zr, r_zl, i,
                      (my_x, my_y, my_z + 1))

        def send_zl(i):
            return mk(prow(my_z + i // P, k, i), s_zl, r_zr, i,
                      (my_x, my_y, my_z - 1))

        def recv_z(d, i):
            ssem, rsem = (s_zr, r_zl) if d == 0 else (s_zl, r_zr)
            return mk(prow(c_of(d, i), k, i), ssem, rsem, i, dev_self)

        def send_z2r(i2):
            return mk(prow(my_z - i2 // P2, kd, i2 % P2), s_z2r, r_z2l, i2,
                      (my_x, my_y, my_z + 1))

        def send_z2l(i2):
            return mk(prow(my_z + i2 // P2, kd, i2 % P2), s_z2l, r_z2r, i2,
                      (my_x, my_y, my_z - 1))

        def recv_z2(d, i2):
            ssem, rsem = (s_z2r, r_z2l) if d == 0 else (s_z2l, r_z2r)
            return mk(prow(c2_of(d, i2), kd, i2 % P2), ssem, rsem, i2,
                      dev_self)

        def xd_send(d, i):
            return mk(prow(c_of(d, i), k, i), s_xd[d], r_xd[d], i, dev_xp)

        def xd_recv(d, i):
            return mk(prow(c_of(d, i), kx, i), s_xd[d], r_xd[d], i, dev_self)

        def yd_send(d, i):
            return mk(prow(c_of(d, i), k, i), s_yd[d], r_yd[d], i, dev_ym)

        def yd_recv(d, i):
            return mk(prow(c_of(d, i), ky, i), s_yd[d], r_yd[d], i, dev_self)

        def yrel_send(d, i):
            return mk(prow(c_of(d, i), kx, i), s_yr[d], r_yr[d], i, dev_ym)

        def yrel_recv(d, i):
            return mk(prow(c_of(d, i), kd, i), s_yr[d], r_yr[d], i, dev_self)

        def xrel_send(d, i):
            return mk(prow(c_of(d, i), ky, i), s_xr[d], r_xr[d], i, dev_xp)

        def xrel_recv(d, i):
            return mk(prow(c_of(d, i), kd, i), s_xr[d], r_xr[d], i, dev_self)

        barrier = pltpu.get_barrier_semaphore()

        @pl.when(my_z >= 1)
        def _():
            pl.semaphore_signal(
                barrier, inc=1, device_id=(my_x, my_y, my_z - 1),
                device_id_type=pl.DeviceIdType.MESH)

        @pl.when(my_z <= N_Z - 2)
        def _():
            pl.semaphore_signal(
                barrier, inc=1, device_id=(my_x, my_y, my_z + 1),
                device_id_type=pl.DeviceIdType.MESH)

        pl.semaphore_signal(barrier, inc=1, device_id=dev_xp,
                            device_id_type=pl.DeviceIdType.MESH)
        pl.semaphore_signal(barrier, inc=1, device_id=dev_ym,
                            device_id_type=pl.DeviceIdType.MESH)
        n_nbrs = 2 + (my_z >= 1).astype(jnp.int32) + (
            my_z <= N_Z - 2).astype(jnp.int32)
        pl.semaphore_wait(barrier, n_nbrs)

        for i in range(P):
            comm_ref[prow(my_z, k, i), :] = x_ref[
                pl.ds(k * Q + (i % P) * Rq, Rq), :].astype(comm_ref.dtype)

            @pl.when(S_R(i))
            def _(i=i):
                send_zr(i).start()

            @pl.when(S_L(i))
            def _(i=i):
                send_zl(i).start()

        for p in range(P2):
            comm_ref[prow(my_z, kd, p), :] = x_ref[
                pl.ds(kd * Q + p * Rq, Rq), :].astype(comm_ref.dtype)

            @pl.when(S2_R(p))
            def _(p=p):
                send_z2r(p).start()

            @pl.when(S2_L(p))
            def _(p=p):
                send_z2l(p).start()

        hbm_store("own", 0)

        conv_units = [(o, p) for o in (1, 2, 3) for p in range(P)]
        n_b = NP - P
        conv_per_iter = [len(conv_units) * (t + 1) // n_b for t in range(n_b)]

        def do_conv(u):
            o, pc = conv_units[u]
            qk = (k + o) % 4
            if pc < P2:
                @pl.when(qk != kd)
                def _():
                    comm_ref[prow(my_z, qk, pc), :] = x_ref[
                        pl.ds(qk * Q + pc * Rq, Rq), :].astype(comm_ref.dtype)
            else:
                comm_ref[prow(my_z, qk, pc), :] = x_ref[
                    pl.ds(qk * Q + pc * Rq, Rq), :].astype(comm_ref.dtype)
            if pc == P - 1:
                hbm_store("own", o)

        conv_done = 0
        for i in range(P, NP):
            j = i - P

            @pl.when(R_of(0, j))
            def _(j=j):
                recv_z(0, j).wait_recv()
                xd_send(0, j).start()
                yd_send(0, j).start()
                if j % P == P - 1:
                    hbm_store("k", 0, j)

            @pl.when(S_R(i))
            def _(i=i):
                send_zr(i).start()

            @pl.when(R_of(1, j))
            def _(j=j):
                recv_z(1, j).wait_recv()
                xd_send(1, j).start()
                yd_send(1, j).start()
                if j % P == P - 1:
                    hbm_store("k", 1, j)

            @pl.when(S_L(i))
            def _(i=i):
                send_zl(i).start()

            if i % P < P2:
                s2 = (i // P) * P2 + (i % P)

                @pl.when(R2_of(0, s2 - P2))
                def _(s2=s2):
                    recv_z2(0, s2 - P2).wait_recv()

                @pl.when(S2_R(s2))
                def _(s2=s2):
                    send_z2r(s2).start()

                @pl.when(R2_of(1, s2 - P2))
                def _(s2=s2):
                    recv_z2(1, s2 - P2).wait_recv()

                @pl.when(S2_L(s2))
                def _(s2=s2):
                    send_z2l(s2).start()

            j2 = i - 2 * P
            if j2 >= 0:
                for d in (0, 1):
                    @pl.when(R_of(d, j2))
                    def _(d=d, j2=j2):
                        xd_recv(d, j2).wait_recv()
                        if j2 % P in C_SET:
                            yrel_send(d, j2).start()
                        if j2 % P == P - 1:
                            hbm_store("kx", d, j2)

                    @pl.when(R_of(d, j2))
                    def _(d=d, j2=j2):
                        yd_recv(d, j2).wait_recv()
                        if j2 % P in B_SET:
                            xrel_send(d, j2).start()
                        if j2 % P == P - 1:
                            hbm_store("ky", d, j2)

            t = i - P
            while conv_done < conv_per_iter[t]:
                do_conv(conv_done)
                conv_done += 1

        for j in range(NP - P, NP):
            for d in (0, 1):
                @pl.when(R_of(d, j))
                def _(d=d, j=j):
                    recv_z(d, j).wait_recv()
                    xd_send(d, j).start()
                    yd_send(d, j).start()
                    if j % P == P - 1:
                        hbm_store("k", d, j)

        for p in range(P2):
            i2 = (N_Z - 2) * P2 + p
            for d in (0, 1):
                @pl.when(R2_of(d, i2))
                def _(d=d, i2=i2):
                    recv_z2(d, i2).wait_recv()

        for j2 in range(max(NP - 2 * P, 0), NP):
            for d in (0, 1):
                @pl.when(R_of(d, j2))
                def _(d=d, j2=j2):
                    xd_recv(d, j2).wait_recv()
                    if j2 % P in C_SET:
                        yrel_send(d, j2).start()
                    if j2 % P == P - 1:
                        hbm_store("kx", d, j2)

                @pl.when(R_of(d, j2))
                def _(d=d, j2=j2):
                    yd_recv(d, j2).wait_recv()
                    if j2 % P in B_SET:
                        xrel_send(d, j2).start()
                    if j2 % P == P - 1:
                        hbm_store("ky", d, j2)

        for i in range(NP):
            for d in (0, 1):
                if i % P in B_SET:
                    @pl.when(R_of(d, i))
                    def _(d=d, i=i):
                        xrel_recv(d, i).wait_recv()
                elif i % P in C_SET:
                    @pl.when(R_of(d, i))
                    def _(d=d, i=i):
                        yrel_recv(d, i).wait_recv()
                        if i % P == P - 1:
                            hbm_store("kd", d, i)

        assert len(sites) == N_SITES, len(sites)
        for idx, (kind, a, j) in enumerate(sites):
            if kind == "own":
                hbm_copy(idx, kind, a, j).wait()
            else:
                @pl.when(R_of(a, j))
                def _(idx=idx, kind=kind, a=a, j=j):
                    hbm_copy(idx, kind, a, j).wait()

        for i in range(NP):
            @pl.when(S_R(i))
            def _(i=i):
                send_zr(i).wait_send()

            @pl.when(S_L(i))
            def _(i=i):
                send_zl(i).wait_send()

            for d in (0, 1):
                @pl.when(R_of(d, i))
                def _(d=d, i=i):
                    xd_send(d, i).wait_send()
                    yd_send(d, i).wait_send()
                    if i % P in C_SET:
                        yrel_send(d, i).wait_send()
                    if i % P in B_SET:
                        xrel_send(d, i).wait_send()

        for i2 in range(NP2):
            @pl.when(S2_R(i2))
            def _(i2=i2):
                send_z2r(i2).wait_send()

            @pl.when(S2_L(i2))
            def _(i2=i2):
                send_z2l(i2).wait_send()

    return pl.pallas_call(
        body,
        out_shape=jax.ShapeDtypeStruct((N_Z * m_per, n), jnp.bfloat16),
        in_specs=[pl.BlockSpec(memory_space=pltpu.VMEM)],
        out_specs=pl.BlockSpec(memory_space=pl.MemorySpace.ANY),
        scratch_shapes=(
            [pltpu.VMEM((N_Z * m_per, n), jnp.bfloat16)]
            + [pltpu.SemaphoreType.DMA((NP,)) for _ in range(4)]
            + [pltpu.SemaphoreType.DMA((NP2,)) for _ in range(4)]
            + [pltpu.SemaphoreType.DMA((NP,)) for _ in range(16)]
            + [pltpu.SemaphoreType.DMA((N_SITES,))]
        ),
        compiler_params=pltpu.CompilerParams(
            collective_id=0, vmem_limit_bytes=100 * 1024 * 1024),
    )(x)


# device time: 144536 ns/iter; 1.1660x vs baseline; 1.0796x over previous
import jax
import jax.numpy as jnp
from jax import lax
from jax.experimental import pallas as pl
from jax.experimental.pallas import tpu as pltpu

N_Z = 4
P = 8
NP = (N_Z - 1) * P
N_SITES = 28


def kernel(x):
    m_per, n = x.shape
    Q = m_per // 4
    Rq = Q // P

    def body(x_ref, out_ref, comm_ref, *sems):
        (s_zr, r_zl, s_zl, r_zr,
         s_xd0, r_xd0, s_xd1, r_xd1,
         s_yd0, r_yd0, s_yd1, r_yd1,
         s_yr0, r_yr0, s_yr1, r_yr1,
         s_xr0, r_xr0, s_xr1, r_xr1, dsem) = sems
        s_xd = (s_xd0, s_xd1); r_xd = (r_xd0, r_xd1)
        s_yd = (s_yd0, s_yd1); r_yd = (r_yd0, r_yd1)
        s_yr = (s_yr0, s_yr1); r_yr = (r_yr0, r_yr1)
        s_xr = (s_xr0, s_xr1); r_xr = (r_xr0, r_xr1)

        my_x = lax.axis_index("x")
        my_y = lax.axis_index("y")
        my_z = lax.axis_index("z")
        yi = my_y % 2
        mate_y = my_y + 1 - 2 * yi
        k = 2 * my_x + yi
        kx = 2 * (1 - my_x) + yi
        ky = 2 * my_x + (1 - yi)
        kd = 3 - k

        dev_self = (my_x, my_y, my_z)
        dev_xp = (1 - my_x, my_y, my_z)
        dev_ym = (my_x, mate_y, my_z)

        def prow(c, qk, i):
            return pl.ds(c * m_per + qk * Q + (i % P) * Rq, Rq)

        def qrow(c, qk):
            return pl.ds(c * m_per + qk * Q, Q)

        def c_of(d, i):
            return (my_z - 1 - i // P) if d == 0 else (my_z + 1 + i // P)

        def R_of(d, i):
            if d == 0:
                return my_z >= 1 + i // P
            return my_z + 1 + i // P <= N_Z - 1

        def S_R(i):
            return (my_z <= N_Z - 2) & (my_z >= i // P)

        def S_L(i):
            return (my_z >= 1) & (my_z + i // P <= N_Z - 1)

        sites = []

        def _site_cq(kind, a, j):
            if kind == "own":
                return my_z, (k + a) % 4
            qk = {"k": k, "kx": kx, "ky": ky, "kd": kd}[kind]
            return c_of(a, j), qk

        def hbm_copy(idx, kind, a, j):
            c, qk = _site_cq(kind, a, j)
            return pltpu.make_async_copy(
                comm_ref.at[qrow(c, qk), :],
                out_ref.at[qrow(c, qk), :],
                dsem.at[idx],
            )

        def hbm_store(kind, a, j=0):
            hbm_copy(len(sites), kind, a, j).start()
            sites.append((kind, a, j))

        def mk(rowslice, ssem, rsem, i, dev):
            return pltpu.make_async_remote_copy(
                src_ref=comm_ref.at[rowslice, :],
                dst_ref=comm_ref.at[rowslice, :],
                send_sem=ssem.at[i],
                recv_sem=rsem.at[i],
                device_id=dev,
                device_id_type=pl.DeviceIdType.MESH,
            )

        def send_zr(i):
            return mk(prow(my_z - i // P, k, i), s_zr, r_zl, i,
                      (my_x, my_y, my_z + 1))

        def send_zl(i):
            return mk(prow(my_z + i // P, k, i), s_zl, r_zr, i,
                      (my_x, my_y, my_z - 1))

        def recv_z(d, i):
            ssem, rsem = (s_zr, r_zl) if d == 0 else (s_zl, r_zr)
            return mk(prow(c_of(d, i), k, i), ssem, rsem, i, dev_self)

        def xd_send(d, i):
            return mk(prow(c_of(d, i), k, i), s_xd[d], r_xd[d], i, dev_xp)

        def xd_recv(d, i):
            return mk(prow(c_of(d, i), kx, i), s_xd[d], r_xd[d], i, dev_self)

        def yd_send(d, i):
            return mk(prow(c_of(d, i), k, i), s_yd[d], r_yd[d], i, dev_ym)

        def yd_recv(d, i):
            return mk(prow(c_of(d, i), ky, i), s_yd[d], r_yd[d], i, dev_self)

        def yrel_send(d, i):
            return mk(prow(c_of(d, i), kx, i), s_yr[d], r_yr[d], i, dev_ym)

        def yrel_recv(d, i):
            return mk(prow(c_of(d, i), kd, i), s_yr[d], r_yr[d], i, dev_self)

        def xrel_send(d, i):
            return mk(prow(c_of(d, i), ky, i), s_xr[d], r_xr[d], i, dev_xp)

        def xrel_recv(d, i):
            return mk(prow(c_of(d, i), kd, i), s_xr[d], r_xr[d], i, dev_self)

        barrier = pltpu.get_barrier_semaphore()

        @pl.when(my_z >= 1)
        def _():
            pl.semaphore_signal(
                barrier, inc=1, device_id=(my_x, my_y, my_z - 1),
                device_id_type=pl.DeviceIdType.MESH)

        @pl.when(my_z <= N_Z - 2)
        def _():
            pl.semaphore_signal(
                barrier, inc=1, device_id=(my_x, my_y, my_z + 1),
                device_id_type=pl.DeviceIdType.MESH)

        pl.semaphore_signal(barrier, inc=1, device_id=dev_xp,
                            device_id_type=pl.DeviceIdType.MESH)
        pl.semaphore_signal(barrier, inc=1, device_id=dev_ym,
                            device_id_type=pl.DeviceIdType.MESH)
        n_nbrs = 2 + (my_z >= 1).astype(jnp.int32) + (
            my_z <= N_Z - 2).astype(jnp.int32)
        pl.semaphore_wait(barrier, n_nbrs)

        for i in range(P):
            comm_ref[prow(my_z, k, i), :] = x_ref[
                pl.ds(k * Q + (i % P) * Rq, Rq), :].astype(comm_ref.dtype)

            @pl.when(S_R(i))
            def _(i=i):
                send_zr(i).start()

            @pl.when(S_L(i))
            def _(i=i):
                send_zl(i).start()

        hbm_store("own", 0)

        conv_units = [(o, p) for o in (1, 2, 3) for p in range(P)]
        n_b = NP - P
        conv_per_iter = [len(conv_units) * (t + 1) // n_b for t in range(n_b)]

        def do_conv(u):
            o, pc = conv_units[u]
            qk = (k + o) % 4
            comm_ref[prow(my_z, qk, pc), :] = x_ref[
                pl.ds(qk * Q + pc * Rq, Rq), :].astype(comm_ref.dtype)
            if pc == P - 1:
                hbm_store("own", o)

        conv_done = 0
        for i in range(P, NP):
            j = i - P

            @pl.when(R_of(0, j))
            def _(j=j):
                recv_z(0, j).wait_recv()
                xd_send(0, j).start()
                yd_send(0, j).start()
                if j % P == P - 1:
                    hbm_store("k", 0, j)

            @pl.when(S_R(i))
            def _(i=i):
                send_zr(i).start()

            @pl.when(R_of(1, j))
            def _(j=j):
                recv_z(1, j).wait_recv()
                xd_send(1, j).start()
                yd_send(1, j).start()
                if j % P == P - 1:
                    hbm_store("k", 1, j)

            @pl.when(S_L(i))
            def _(i=i):
                send_zl(i).start()

            j2 = i - 2 * P
            if j2 >= 0:
                for d in (0, 1):
                    @pl.when(R_of(d, j2))
                    def _(d=d, j2=j2):
                        xd_recv(d, j2).wait_recv()
                        if j2 % 2 == 0:
                            yrel_send(d, j2).start()
                        if j2 % P == P - 1:
                            hbm_store("kx", d, j2)

                    @pl.when(R_of(d, j2))
                    def _(d=d, j2=j2):
                        yd_recv(d, j2).wait_recv()
                        if j2 % 2 == 1:
                            xrel_send(d, j2).start()
                        if j2 % P == P - 1:
                            hbm_store("ky", d, j2)

            t = i - P
            while conv_done < conv_per_iter[t]:
                do_conv(conv_done)
                conv_done += 1

        for j in range(NP - P, NP):
            for d in (0, 1):
                @pl.when(R_of(d, j))
                def _(d=d, j=j):
                    recv_z(d, j).wait_recv()
                    xd_send(d, j).start()
                    yd_send(d, j).start()
                    if j % P == P - 1:
                        hbm_store("k", d, j)

        for j2 in range(max(NP - 2 * P, 0), NP):
            for d in (0, 1):
                @pl.when(R_of(d, j2))
                def _(d=d, j2=j2):
                    xd_recv(d, j2).wait_recv()
                    if j2 % 2 == 0:
                        yrel_send(d, j2).start()
                    if j2 % P == P - 1:
                        hbm_store("kx", d, j2)

                @pl.when(R_of(d, j2))
                def _(d=d, j2=j2):
                    yd_recv(d, j2).wait_recv()
                    if j2 % 2 == 1:
                        xrel_send(d, j2).start()
                    if j2 % P == P - 1:
                        hbm_store("ky", d, j2)

        for i in range(NP):
            for d in (0, 1):
                if i % 2 == 0:
                    @pl.when(R_of(d, i))
                    def _(d=d, i=i):
                        yrel_recv(d, i).wait_recv()
                else:
                    @pl.when(R_of(d, i))
                    def _(d=d, i=i):
                        xrel_recv(d, i).wait_recv()
                        if i % P == P - 1:
                            hbm_store("kd", d, i)

        assert len(sites) == N_SITES, len(sites)
        for idx, (kind, a, j) in enumerate(sites):
            if kind == "own":
                hbm_copy(idx, kind, a, j).wait()
            else:
                @pl.when(R_of(a, j))
                def _(idx=idx, kind=kind, a=a, j=j):
                    hbm_copy(idx, kind, a, j).wait()

        for i in range(NP):
            @pl.when(S_R(i))
            def _(i=i):
                send_zr(i).wait_send()

            @pl.when(S_L(i))
            def _(i=i):
                send_zl(i).wait_send()

            for d in (0, 1):
                @pl.when(R_of(d, i))
                def _(d=d, i=i):
                    xd_send(d, i).wait_send()
                    yd_send(d, i).wait_send()
                    if i % 2 == 0:
                        yrel_send(d, i).wait_send()
                    else:
                        xrel_send(d, i).wait_send()

    return pl.pallas_call(
        body,
        out_shape=jax.ShapeDtypeStruct((N_Z * m_per, n), jnp.bfloat16),
        in_specs=[pl.BlockSpec(memory_space=pltpu.VMEM)],
        out_specs=pl.BlockSpec(memory_space=pl.MemorySpace.ANY),
        scratch_shapes=(
            [pltpu.VMEM((N_Z * m_per, n), jnp.bfloat16)]
            + [pltpu.SemaphoreType.DMA((NP,)) for _ in range(20)]
            + [pltpu.SemaphoreType.DMA((N_SITES,))]
        ),
        compiler_params=pltpu.CompilerParams(collective_id=0),
    )(x)
